# Initial kernel scaffold; baseline (speedup 1.0000x reference)
import jax
import jax.numpy as jnp
from jax.experimental import pallas as pl


def kernel(tokens, segments, word_emb, pos_emb, type_emb, gamma, beta):
    raise NotImplementedError("write your pallas kernel here")


# speedup vs baseline: 1.9537x; 1.9537x over previous
import functools

import jax
import jax.numpy as jnp
from jax import lax
from jax.experimental import pallas as pl
from jax.experimental.pallas import tpu as pltpu
from jax.experimental.pallas import tpu_sc as plsc

_HID = 128
_NJ = _HID // 16
_C = 128
_NW = 32


_GDN = lax.GatherDimensionNumbers(
    offset_dims=(), collapsed_slice_dims=(0,), start_index_map=(0,))


def _lane_allreduce(v):
  for k in (8, 4, 2, 1):
    perm = (jnp.arange(16, dtype=jnp.int32) ^ k)[:, None]
    v = v + lax.gather(v, perm, _GDN, (1,),
                       mode=lax.GatherScatterMode.PROMISE_IN_BOUNDS)
  return v


def _ln_token(rows_v, comb_v, gb_v, t, cix):
  xs = [rows_v[t, pl.ds(16 * j, 16)] + comb_v[cix, pl.ds(16 * j, 16)]
        for j in range(_NJ)]
  s = xs[0]
  for j in range(1, _NJ):
    s = s + xs[j]
  ss = xs[0] * xs[0]
  for j in range(1, _NJ):
    ss = ss + xs[j] * xs[j]
  mean_v = _lane_allreduce(s) * (1.0 / _HID)
  a_v = _lane_allreduce(ss) * (1.0 / _HID) - mean_v * mean_v + 1e-12
  ii = lax.bitcast_convert_type(a_v, jnp.int32)
  ii = jnp.int32(0x5F3759DF) - lax.shift_right_logical(ii, 1)
  y = lax.bitcast_convert_type(ii, jnp.float32)
  for _ in range(3):
    y = y * (1.5 - 0.5 * a_v * y * y)
  for j in range(_NJ):
    g = gb_v[0, pl.ds(16 * j, 16)]
    b = gb_v[1, pl.ds(16 * j, 16)]
    rows_v[t, pl.ds(16 * j, 16)] = (xs[j] - mean_v) * y * g + b


def _embed_ln(tokens_flat, segments_flat, word_emb, comb, gamma, beta, L):
  N = tokens_flat.shape[0]
  n_chunks = N // _C
  per_w = n_chunks // _NW

  mesh = plsc.VectorSubcoreMesh(core_axis_name="c", subcore_axis_name="s")

  @functools.partial(
      pl.kernel,
      out_type=jax.ShapeDtypeStruct((N, _HID), jnp.float32),
      mesh=mesh,
      scratch_types=[
          pltpu.VMEM((_C,), jnp.int32),
          pltpu.VMEM((_C + 16,), jnp.int32),
          pltpu.VMEM((_C, _HID), jnp.float32),
          pltpu.VMEM((2 * L, _HID), jnp.float32),
          pltpu.VMEM((2, _HID), jnp.float32),
          pltpu.SemaphoreType.DMA,
      ],
  )
  def body(tok_hbm, seg_hbm, word_hbm, comb_hbm, gamma_hbm, beta_hbm, out_hbm,
           tok_v, seg_v, rows_v, comb_v, gb_v, sem):
    wid = lax.axis_index("s") * 2 + lax.axis_index("c")
    pltpu.sync_copy(comb_hbm, comb_v)
    pltpu.sync_copy(gamma_hbm, gb_v.at[0])
    pltpu.sync_copy(beta_hbm, gb_v.at[1])

    def chunk_body(i, carry):
      base = (wid * per_w + i) * _C
      pltpu.sync_copy(tok_hbm.at[pl.ds(base, _C)], tok_v)
      pltpu.sync_copy(seg_hbm.at[pl.ds(base, _C)], seg_v.at[pl.ds(0, _C)])
      pltpu.async_copy(word_hbm.at[tok_v], rows_v, sem).wait()

      def tok_body(t, c2):
        seg = seg_v[pl.ds(t, 16)][0]
        pos = lax.rem(base + t, L)
        cix = 2 * pos + seg
        _ln_token(rows_v, comb_v, gb_v, t, cix)
        return c2

      lax.fori_loop(0, _C, tok_body, 0)
      pltpu.async_copy(rows_v, out_hbm.at[pl.ds(base, _C)], sem).wait()
      return carry

    lax.fori_loop(0, per_w, chunk_body, 0)

  return body(tokens_flat, segments_flat, word_emb, comb, gamma, beta)


def kernel(tokens, segments, word_emb, pos_emb, type_emb, gamma, beta):
  B, L = tokens.shape
  comb = (pos_emb[:L, None, :] + type_emb[None, :, :]).reshape(2 * L, _HID)
  out = _embed_ln(tokens.astype(jnp.int32).reshape(-1),
                  segments.astype(jnp.int32).reshape(-1),
                  word_emb, comb, gamma, beta, L)
  return out.reshape(B, L, _HID)


# speedup vs baseline: 3.7487x; 1.9188x over previous
import functools

import jax
import jax.numpy as jnp
from jax import lax
from jax.experimental import pallas as pl
from jax.experimental.pallas import tpu as pltpu
from jax.experimental.pallas import tpu_sc as plsc

_HID = 128
_NJ = _HID // 16
_C = 128
_NW = 32
_NBUF = 3
_UNROLL = 4

_GDN = lax.GatherDimensionNumbers(
    offset_dims=(), collapsed_slice_dims=(0,), start_index_map=(0,))


def _lane_allreduce(v):
  for k in (8, 4, 2, 1):
    perm = (jnp.arange(16, dtype=jnp.int32) ^ k)[:, None]
    v = v + lax.gather(v, perm, _GDN, (1,),
                       mode=lax.GatherScatterMode.PROMISE_IN_BOUNDS)
  return v


def _ln_token(rows_b, crows_b, gs, bs, t):
  x = [rows_b[t, pl.ds(16 * j, 16)] + crows_b[t, pl.ds(16 * j, 16)]
       for j in range(_NJ)]
  s = ((x[0] + x[1]) + (x[2] + x[3])) + ((x[4] + x[5]) + (x[6] + x[7]))
  q = [v * v for v in x]
  ss = ((q[0] + q[1]) + (q[2] + q[3])) + ((q[4] + q[5]) + (q[6] + q[7]))
  mean_v = _lane_allreduce(s) * (1.0 / _HID)
  a_v = _lane_allreduce(ss) * (1.0 / _HID) - mean_v * mean_v + 1e-12
  ii = lax.bitcast_convert_type(a_v, jnp.int32)
  ii = jnp.int32(0x5F3759DF) - lax.shift_right_logical(ii, 1)
  y = lax.bitcast_convert_type(ii, jnp.float32)
  for _ in range(3):
    y = y * (1.5 - 0.5 * a_v * y * y)
  for j in range(_NJ):
    rows_b[t, pl.ds(16 * j, 16)] = (x[j] - mean_v) * y * gs[j] + bs[j]


def _embed_ln(tokens_flat, segments_flat, word_emb, comb, gamma, beta, L):
  N = tokens_flat.shape[0]
  n_chunks = N // _C
  per_w = n_chunks // _NW

  mesh = plsc.VectorSubcoreMesh(core_axis_name="c", subcore_axis_name="s")

  @functools.partial(
      pl.kernel,
      out_type=jax.ShapeDtypeStruct((N, _HID), jnp.float32),
      mesh=mesh,
      scratch_types=[
          pltpu.VMEM((_NBUF, _C), jnp.int32),
          pltpu.VMEM((_NBUF, _C), jnp.int32),
          pltpu.VMEM((_NBUF, _C), jnp.int32),
          pltpu.VMEM((_NBUF, _C, _HID), jnp.float32),
          pltpu.VMEM((_NBUF, _C, _HID), jnp.float32),
          pltpu.VMEM((2, _HID), jnp.float32),
          pltpu.SemaphoreType.DMA((_NBUF,)),
          pltpu.SemaphoreType.DMA((_NBUF,)),
      ],
  )
  def body(tok_hbm, seg_hbm, word_hbm, comb_hbm, gamma_hbm, beta_hbm, out_hbm,
           tok_v, seg_v, cix_v, rows_v, crows_v, gb_v, in_sem, out_sem):
    wid = lax.axis_index("s") * 2 + lax.axis_index("c")
    w0 = wid * per_w
    pltpu.sync_copy(gamma_hbm, gb_v.at[0])
    pltpu.sync_copy(beta_hbm, gb_v.at[1])
    iota = lax.iota(jnp.int32, 16)

    def issue(c):
      b = lax.rem(c, _NBUF)
      base = (w0 + c) * _C
      pltpu.sync_copy(tok_hbm.at[pl.ds(base, _C)], tok_v.at[b])
      pltpu.sync_copy(seg_hbm.at[pl.ds(base, _C)], seg_v.at[b])
      for g in range(_C // 16):
        pos = lax.rem(base + g * 16 + iota, L)
        cix_v[b, pl.ds(g * 16, 16)] = 2 * pos + seg_v[b, pl.ds(g * 16, 16)]
      pltpu.async_copy(word_hbm.at[tok_v.at[b]], rows_v.at[b], in_sem.at[b])
      pltpu.async_copy(comb_hbm.at[cix_v.at[b]], crows_v.at[b], in_sem.at[b])

    issue(0)
    issue(1)

    def loop_body(i, carry):
      gs, bs = carry
      b = lax.rem(i, _NBUF)
      base = (w0 + i) * _C
      pltpu.make_async_copy(word_hbm.at[pl.ds(0, _C)], rows_v.at[b],
                            in_sem.at[b]).wait()
      pltpu.make_async_copy(comb_hbm.at[pl.ds(0, _C)], crows_v.at[b],
                            in_sem.at[b]).wait()
      rows_b = rows_v.at[b]
      crows_b = crows_v.at[b]

      def tok_group(g, c2):
        for u in range(_UNROLL):
          _ln_token(rows_b, crows_b, gs, bs, g * _UNROLL + u)
        return c2

      lax.fori_loop(0, _C // _UNROLL, tok_group, 0)
      pltpu.async_copy(rows_v.at[b], out_hbm.at[pl.ds(base, _C)],
                       out_sem.at[b])

      @pl.when(i + 2 < per_w)
      def _():
        b2 = lax.rem(i + 2, _NBUF)

        @pl.when(i >= 1)
        def _():
          pltpu.make_async_copy(rows_v.at[b2], out_hbm.at[pl.ds(0, _C)],
                                out_sem.at[b2]).wait()

        issue(i + 2)

      return gs, bs

    gs0 = tuple(gb_v[0, pl.ds(16 * j, 16)] for j in range(_NJ))
    bs0 = tuple(gb_v[1, pl.ds(16 * j, 16)] for j in range(_NJ))
    lax.fori_loop(0, per_w, loop_body, (gs0, bs0))
    for k in range(_NBUF):
      pltpu.make_async_copy(rows_v.at[k], out_hbm.at[pl.ds(0, _C)],
                            out_sem.at[k]).wait()

  return body(tokens_flat, segments_flat, word_emb, comb, gamma, beta)


def kernel(tokens, segments, word_emb, pos_emb, type_emb, gamma, beta):
  B, L = tokens.shape
  comb = (pos_emb[:L, None, :] + type_emb[None, :, :]).reshape(2 * L, _HID)
  out = _embed_ln(tokens.astype(jnp.int32).reshape(-1),
                  segments.astype(jnp.int32).reshape(-1),
                  word_emb, comb, gamma, beta, L)
  return out.reshape(B, L, _HID)


# speedup vs baseline: 4.0400x; 1.0777x over previous
import functools

import jax
import jax.numpy as jnp
from jax import lax
from jax.experimental import pallas as pl
from jax.experimental.pallas import tpu as pltpu
from jax.experimental.pallas import tpu_sc as plsc

_HID = 128
_NJ = _HID // 16
_C = 128
_NW = 32
_NBUF = 3
_UNROLL = 8

_GDN = lax.GatherDimensionNumbers(
    offset_dims=(), collapsed_slice_dims=(0,), start_index_map=(0,))


def _lane_allreduce(v):
  for k in (8, 4, 2, 1):
    perm = (jnp.arange(16, dtype=jnp.int32) ^ k)[:, None]
    v = v + lax.gather(v, perm, _GDN, (1,),
                       mode=lax.GatherScatterMode.PROMISE_IN_BOUNDS)
  return v


def _ln_token(rows_b, crows_b, gs, bs, t):
  x = [rows_b[t, pl.ds(16 * j, 16)] + crows_b[t, pl.ds(16 * j, 16)]
       for j in range(_NJ)]
  s = ((x[0] + x[1]) + (x[2] + x[3])) + ((x[4] + x[5]) + (x[6] + x[7]))
  q = [v * v for v in x]
  ss = ((q[0] + q[1]) + (q[2] + q[3])) + ((q[4] + q[5]) + (q[6] + q[7]))
  mean_v = _lane_allreduce(s) * (1.0 / _HID)
  a_v = _lane_allreduce(ss) * (1.0 / _HID) - mean_v * mean_v + 1e-12
  ii = lax.bitcast_convert_type(a_v, jnp.int32)
  ii = jnp.int32(0x5F3759DF) - lax.shift_right_logical(ii, 1)
  y = lax.bitcast_convert_type(ii, jnp.float32)
  for _ in range(2):
    y = y * (1.5 - 0.5 * a_v * y * y)
  for j in range(_NJ):
    rows_b[t, pl.ds(16 * j, 16)] = (x[j] - mean_v) * y * gs[j] + bs[j]


def _embed_ln(tokens_flat, segments_flat, word_emb, comb, gamma, beta, L):
  N = tokens_flat.shape[0]
  n_chunks = N // _C
  per_w = n_chunks // _NW

  mesh = plsc.VectorSubcoreMesh(core_axis_name="c", subcore_axis_name="s")

  @functools.partial(
      pl.kernel,
      out_type=jax.ShapeDtypeStruct((N, _HID), jnp.float32),
      mesh=mesh,
      scratch_types=[
          pltpu.VMEM((_NBUF, _C), jnp.int32),
          pltpu.VMEM((_NBUF, _C), jnp.int32),
          pltpu.VMEM((_NBUF, _C), jnp.int32),
          pltpu.VMEM((_NBUF, _C, _HID), jnp.float32),
          pltpu.VMEM((_NBUF, _C, _HID), jnp.float32),
          pltpu.VMEM((2, _HID), jnp.float32),
          pltpu.SemaphoreType.DMA((_NBUF,)),
          pltpu.SemaphoreType.DMA((_NBUF,)),
      ],
  )
  def body(tok_hbm, seg_hbm, word_hbm, comb_hbm, gamma_hbm, beta_hbm, out_hbm,
           tok_v, seg_v, cix_v, rows_v, crows_v, gb_v, in_sem, out_sem):
    wid = lax.axis_index("s") * 2 + lax.axis_index("c")
    w0 = wid * per_w
    pltpu.sync_copy(gamma_hbm, gb_v.at[0])
    pltpu.sync_copy(beta_hbm, gb_v.at[1])
    iota = lax.iota(jnp.int32, 16)

    def issue(c):
      b = lax.rem(c, _NBUF)
      base = (w0 + c) * _C
      pltpu.sync_copy(tok_hbm.at[pl.ds(base, _C)], tok_v.at[b])
      pltpu.sync_copy(seg_hbm.at[pl.ds(base, _C)], seg_v.at[b])
      for g in range(_C // 16):
        pos = lax.rem(base + g * 16 + iota, L)
        cix_v[b, pl.ds(g * 16, 16)] = 2 * pos + seg_v[b, pl.ds(g * 16, 16)]
      pltpu.async_copy(word_hbm.at[tok_v.at[b]], rows_v.at[b], in_sem.at[b])
      pltpu.async_copy(comb_hbm.at[cix_v.at[b]], crows_v.at[b], in_sem.at[b])

    issue(0)
    issue(1)

    def loop_body(i, carry):
      gs, bs = carry
      b = lax.rem(i, _NBUF)
      base = (w0 + i) * _C
      pltpu.make_async_copy(word_hbm.at[pl.ds(0, _C)], rows_v.at[b],
                            in_sem.at[b]).wait()
      pltpu.make_async_copy(comb_hbm.at[pl.ds(0, _C)], crows_v.at[b],
                            in_sem.at[b]).wait()
      rows_b = rows_v.at[b]
      crows_b = crows_v.at[b]

      def tok_group(g, c2):
        for u in range(_UNROLL):
          _ln_token(rows_b, crows_b, gs, bs, g * _UNROLL + u)
        return c2

      lax.fori_loop(0, _C // _UNROLL, tok_group, 0)
      pltpu.async_copy(rows_v.at[b], out_hbm.at[pl.ds(base, _C)],
                       out_sem.at[b])

      @pl.when(i + 2 < per_w)
      def _():
        b2 = lax.rem(i + 2, _NBUF)

        @pl.when(i >= 1)
        def _():
          pltpu.make_async_copy(rows_v.at[b2], out_hbm.at[pl.ds(0, _C)],
                                out_sem.at[b2]).wait()

        issue(i + 2)

      return gs, bs

    gs0 = tuple(gb_v[0, pl.ds(16 * j, 16)] for j in range(_NJ))
    bs0 = tuple(gb_v[1, pl.ds(16 * j, 16)] for j in range(_NJ))
    lax.fori_loop(0, per_w, loop_body, (gs0, bs0))
    for k in range(_NBUF):
      pltpu.make_async_copy(rows_v.at[k], out_hbm.at[pl.ds(0, _C)],
                            out_sem.at[k]).wait()

  return body(tokens_flat, segments_flat, word_emb, comb, gamma, beta)


def kernel(tokens, segments, word_emb, pos_emb, type_emb, gamma, beta):
  B, L = tokens.shape
  comb = (pos_emb[:L, None, :] + type_emb[None, :, :]).reshape(2 * L, _HID)
  out = _embed_ln(tokens.astype(jnp.int32).reshape(-1),
                  segments.astype(jnp.int32).reshape(-1),
                  word_emb, comb, gamma, beta, L)
  return out.reshape(B, L, _HID)


# speedup vs baseline: 4.5168x; 1.1180x over previous
import functools

import jax
import jax.numpy as jnp
from jax import lax
from jax.experimental import pallas as pl
from jax.experimental.pallas import tpu as pltpu
from jax.experimental.pallas import tpu_sc as plsc

_HID = 128
_NJ = _HID // 16
_C = 128
_NW = 32
_NBUF = 3
_UNROLL = 8

_GDN = lax.GatherDimensionNumbers(
    offset_dims=(), collapsed_slice_dims=(0,), start_index_map=(0,))


def _lane_allreduce(v):
  for k in (8, 4, 2, 1):
    perm = (jnp.arange(16, dtype=jnp.int32) ^ k)[:, None]
    v = v + lax.gather(v, perm, _GDN, (1,),
                       mode=lax.GatherScatterMode.PROMISE_IN_BOUNDS)
  return v


def _ln_token(rows_b, crows_b, gs, bs, t):
  x = [rows_b[t, pl.ds(16 * j, 16)] + crows_b[t, pl.ds(16 * j, 16)]
       for j in range(_NJ)]
  s = ((x[0] + x[1]) + (x[2] + x[3])) + ((x[4] + x[5]) + (x[6] + x[7]))
  q = [v * v for v in x]
  ss = ((q[0] + q[1]) + (q[2] + q[3])) + ((q[4] + q[5]) + (q[6] + q[7]))
  mean_v = _lane_allreduce(s) * (1.0 / _HID)
  a_v = _lane_allreduce(ss) * (1.0 / _HID) - mean_v * mean_v + 1e-12
  ii = lax.bitcast_convert_type(a_v, jnp.int32)
  ii = jnp.int32(0x5F3759DF) - lax.shift_right_logical(ii, 1)
  y = lax.bitcast_convert_type(ii, jnp.float32)
  for _ in range(2):
    y = y * (1.5 - 0.5 * a_v * y * y)
  for j in range(_NJ):
    rows_b[t, pl.ds(16 * j, 16)] = (x[j] - mean_v) * y * gs[j] + bs[j]


def _embed_ln(tokens_flat, segments_flat, word_emb, comb, gamma, beta, L):
  N = tokens_flat.shape[0]
  n_chunks = N // _C
  per_w = n_chunks // _NW
  npw = per_w * _C

  mesh = plsc.VectorSubcoreMesh(core_axis_name="c", subcore_axis_name="s")

  @functools.partial(
      pl.kernel,
      out_type=jax.ShapeDtypeStruct((N, _HID), jnp.float32),
      mesh=mesh,
      scratch_types=[
          pltpu.VMEM((npw,), jnp.int32),
          pltpu.VMEM((npw,), jnp.int32),
          pltpu.VMEM((npw,), jnp.int32),
          pltpu.VMEM((_NBUF, _C, _HID), jnp.float32),
          pltpu.VMEM((_NBUF, _C, _HID), jnp.float32),
          pltpu.VMEM((2, _HID), jnp.float32),
          pltpu.SemaphoreType.DMA((_NBUF,)),
          pltpu.SemaphoreType.DMA((_NBUF,)),
      ],
  )
  def body(tok_hbm, seg_hbm, word_hbm, comb_hbm, gamma_hbm, beta_hbm, out_hbm,
           tok_v, seg_v, cix_v, rows_v, crows_v, gb_v, in_sem, out_sem):
    wid = lax.axis_index("s") * 2 + lax.axis_index("c")
    w0 = wid * npw
    pltpu.sync_copy(gamma_hbm, gb_v.at[0])
    pltpu.sync_copy(beta_hbm, gb_v.at[1])
    pltpu.sync_copy(tok_hbm.at[pl.ds(w0, npw)], tok_v)
    pltpu.sync_copy(seg_hbm.at[pl.ds(w0, npw)], seg_v)
    iota = lax.iota(jnp.int32, 16)

    def cix_group(g, c2):
      pos = lax.rem(w0 + g * 16 + iota, L)
      cix_v[pl.ds(g * 16, 16)] = 2 * pos + seg_v[pl.ds(g * 16, 16)]
      return c2

    lax.fori_loop(0, npw // 16, cix_group, 0)

    def issue(c):
      b = lax.rem(c, _NBUF)
      off = c * _C
      pltpu.async_copy(word_hbm.at[tok_v.at[pl.ds(off, _C)]], rows_v.at[b],
                       in_sem.at[b])
      pltpu.async_copy(comb_hbm.at[cix_v.at[pl.ds(off, _C)]], crows_v.at[b],
                       in_sem.at[b])

    issue(0)
    issue(1)

    def loop_body(i, carry):
      gs, bs = carry
      b = lax.rem(i, _NBUF)
      base = w0 + i * _C
      pltpu.make_async_copy(word_hbm.at[pl.ds(0, _C)], rows_v.at[b],
                            in_sem.at[b]).wait()
      pltpu.make_async_copy(comb_hbm.at[pl.ds(0, _C)], crows_v.at[b],
                            in_sem.at[b]).wait()
      rows_b = rows_v.at[b]
      crows_b = crows_v.at[b]

      def tok_group(g, c2):
        for u in range(_UNROLL):
          _ln_token(rows_b, crows_b, gs, bs, g * _UNROLL + u)
        return c2

      lax.fori_loop(0, _C // _UNROLL, tok_group, 0)
      pltpu.async_copy(rows_v.at[b], out_hbm.at[pl.ds(base, _C)],
                       out_sem.at[b])

      @pl.when(i + 2 < per_w)
      def _():
        b2 = lax.rem(i + 2, _NBUF)

        @pl.when(i >= 1)
        def _():
          pltpu.make_async_copy(rows_v.at[b2], out_hbm.at[pl.ds(0, _C)],
                                out_sem.at[b2]).wait()

        issue(i + 2)

      return gs, bs

    gs0 = tuple(gb_v[0, pl.ds(16 * j, 16)] for j in range(_NJ))
    bs0 = tuple(gb_v[1, pl.ds(16 * j, 16)] for j in range(_NJ))
    lax.fori_loop(0, per_w, loop_body, (gs0, bs0))
    for k in range(_NBUF):
      pltpu.make_async_copy(rows_v.at[k], out_hbm.at[pl.ds(0, _C)],
                            out_sem.at[k]).wait()

  return body(tokens_flat, segments_flat, word_emb, comb, gamma, beta)


def kernel(tokens, segments, word_emb, pos_emb, type_emb, gamma, beta):
  B, L = tokens.shape
  comb = (pos_emb[:L, None, :] + type_emb[None, :, :]).reshape(2 * L, _HID)
  out = _embed_ln(tokens.astype(jnp.int32).reshape(-1),
                  segments.astype(jnp.int32).reshape(-1),
                  word_emb, comb, gamma, beta, L)
  return out.reshape(B, L, _HID)


# speedup vs baseline: 4.5310x; 1.0031x over previous
import functools

import jax
import jax.numpy as jnp
from jax import lax
from jax.experimental import pallas as pl
from jax.experimental.pallas import tpu as pltpu
from jax.experimental.pallas import tpu_sc as plsc

_HID = 128
_NJ = _HID // 16
_C = 128
_NW = 32
_NBUF = 3
_G = 16

_GDN = lax.GatherDimensionNumbers(
    offset_dims=(), collapsed_slice_dims=(0,), start_index_map=(0,))


def _lane_allreduce(v):
  for k in (8, 4, 2, 1):
    perm = (jnp.arange(16, dtype=jnp.int32) ^ k)[:, None]
    v = v + lax.gather(v, perm, _GDN, (1,),
                       mode=lax.GatherScatterMode.PROMISE_IN_BOUNDS)
  return v


def _ln_token(rows_b, comb_v, gs, bs, t, cix):
  x = [rows_b[t, pl.ds(16 * j, 16)] + comb_v[cix, pl.ds(16 * j, 16)]
       for j in range(_NJ)]
  s = ((x[0] + x[1]) + (x[2] + x[3])) + ((x[4] + x[5]) + (x[6] + x[7]))
  q = [v * v for v in x]
  ss = ((q[0] + q[1]) + (q[2] + q[3])) + ((q[4] + q[5]) + (q[6] + q[7]))
  mean_v = _lane_allreduce(s) * (1.0 / _HID)
  a_v = _lane_allreduce(ss) * (1.0 / _HID) - mean_v * mean_v + 1e-12
  ii = lax.bitcast_convert_type(a_v, jnp.int32)
  ii = jnp.int32(0x5F3759DF) - lax.shift_right_logical(ii, 1)
  y = lax.bitcast_convert_type(ii, jnp.float32)
  for _ in range(2):
    y = y * (1.5 - 0.5 * a_v * y * y)
  for j in range(_NJ):
    rows_b[t, pl.ds(16 * j, 16)] = (x[j] - mean_v) * y * gs[j] + bs[j]


def _embed_ln(tokens_flat, segments_flat, word_emb, comb, gamma, beta, L):
  N = tokens_flat.shape[0]
  n_chunks = N // _C
  per_w = n_chunks // _NW
  npw = per_w * _C

  mesh = plsc.VectorSubcoreMesh(core_axis_name="c", subcore_axis_name="s")

  @functools.partial(
      pl.kernel,
      out_type=jax.ShapeDtypeStruct((N, _HID), jnp.float32),
      mesh=mesh,
      scratch_types=[
          pltpu.VMEM((npw,), jnp.int32),
          pltpu.VMEM((npw,), jnp.int32),
          pltpu.VMEM((npw,), jnp.int32),
          pltpu.VMEM((_NBUF, _C, _HID), jnp.float32),
          pltpu.VMEM((2 * L, _HID), jnp.float32),
          pltpu.VMEM((2, _HID), jnp.float32),
          pltpu.SemaphoreType.DMA((_NBUF,)),
          pltpu.SemaphoreType.DMA((_NBUF,)),
      ],
  )
  def body(tok_hbm, seg_hbm, word_hbm, comb_hbm, gamma_hbm, beta_hbm, out_hbm,
           tok_v, seg_v, cix_v, rows_v, comb_v, gb_v, in_sem, out_sem):
    wid = lax.axis_index("s") * 2 + lax.axis_index("c")
    w0 = wid * npw
    pltpu.sync_copy(gamma_hbm, gb_v.at[0])
    pltpu.sync_copy(beta_hbm, gb_v.at[1])
    pltpu.sync_copy(comb_hbm, comb_v)
    pltpu.sync_copy(tok_hbm.at[pl.ds(w0, npw)], tok_v)
    pltpu.sync_copy(seg_hbm.at[pl.ds(w0, npw)], seg_v)
    iota = lax.iota(jnp.int32, 16)

    def cix_group(g, c2):
      pos = lax.rem(w0 + g * 16 + iota, L)
      cix_v[pl.ds(g * 16, 16)] = 2 * pos + seg_v[pl.ds(g * 16, 16)]
      return c2

    lax.fori_loop(0, npw // 16, cix_group, 0)

    def issue(c):
      b = lax.rem(c, _NBUF)
      pltpu.async_copy(word_hbm.at[tok_v.at[pl.ds(c * _C, _C)]], rows_v.at[b],
                       in_sem.at[b])

    issue(0)
    issue(1)

    def loop_body(i, carry):
      gs, bs = carry
      b = lax.rem(i, _NBUF)
      base = w0 + i * _C
      pltpu.make_async_copy(word_hbm.at[pl.ds(0, _C)], rows_v.at[b],
                            in_sem.at[b]).wait()
      rows_b = rows_v.at[b]

      def tok_group(g, c2):
        cvec = cix_v[pl.ds(i * _C + g * _G, 16)]
        for u in range(_G):
          _ln_token(rows_b, comb_v, gs, bs, g * _G + u, cvec[u])
        return c2

      lax.fori_loop(0, _C // _G, tok_group, 0)
      pltpu.async_copy(rows_v.at[b], out_hbm.at[pl.ds(base, _C)],
                       out_sem.at[b])

      @pl.when(i + 2 < per_w)
      def _():
        b2 = lax.rem(i + 2, _NBUF)

        @pl.when(i >= 1)
        def _():
          pltpu.make_async_copy(rows_v.at[b2], out_hbm.at[pl.ds(0, _C)],
                                out_sem.at[b2]).wait()

        issue(i + 2)

      return gs, bs

    gs0 = tuple(gb_v[0, pl.ds(16 * j, 16)] for j in range(_NJ))
    bs0 = tuple(gb_v[1, pl.ds(16 * j, 16)] for j in range(_NJ))
    lax.fori_loop(0, per_w, loop_body, (gs0, bs0))
    for k in range(_NBUF):
      pltpu.make_async_copy(rows_v.at[k], out_hbm.at[pl.ds(0, _C)],
                            out_sem.at[k]).wait()

  return body(tokens_flat, segments_flat, word_emb, comb, gamma, beta)


def kernel(tokens, segments, word_emb, pos_emb, type_emb, gamma, beta):
  B, L = tokens.shape
  comb = (pos_emb[:L, None, :] + type_emb[None, :, :]).reshape(2 * L, _HID)
  out = _embed_ln(tokens.astype(jnp.int32).reshape(-1),
                  segments.astype(jnp.int32).reshape(-1),
                  word_emb, comb, gamma, beta, L)
  return out.reshape(B, L, _HID)
